# baseline (speedup 1.0000x reference)
import functools
import numpy as np
import jax
import jax.numpy as jnp
from jax import lax
from jax.experimental import pallas as pl
from jax.experimental.pallas import tpu as pltpu
from jax.experimental.pallas import tpu_sc as plsc

_P = 0.1
_S = 1.0 - 2.0 * _P
_tmp = np.tile(np.array([_P, _S, _P], dtype=np.float32), (3, 3, 1))
_tmp[2, 1] = np.array([0.0, _S / (_P + _S), _P / (_P + _S)], dtype=np.float32)
_LOGITS = np.where(_tmp > 0, np.log(np.maximum(_tmp, 1e-30)), -1e30).astype(np.float32)
_LN = _LOGITS[0, 0]
_LS = _LOGITS[2, 1]

_LANES = 16
_CHUNK = 256
_NCHUNK = 16
_ICHUNK = 32


def _chain_body(g_hbm, s_hbm, gbuf0, gbuf1, sbuf0, sbuf1,
                gsem0, gsem1, ssem0, ssem1):
    cid = lax.axis_index("c")
    sid = lax.axis_index("s")

    @pl.when(sid == 0)
    def _():
        gbufs = (gbuf0, gbuf1)
        sbufs = (sbuf0, sbuf1)
        gsems = (gsem0, gsem1)
        ssems = (ssem0, ssem1)

        ln0 = jnp.float32(_LN[0])
        ln1 = jnp.float32(_LN[1])
        ln2 = jnp.float32(_LN[2])
        ls1 = jnp.float32(_LS[1])
        ls2 = jnp.float32(_LS[2])

        lane = lax.iota(jnp.int32, _LANES)
        gcol = (cid * _LANES + lane) * 3
        srow = lane

        def step(j, carry, gbuf, sbuf):
            p1, p2 = carry
            rowj = jnp.full((_LANES,), j, jnp.int32)
            g0 = plsc.load_gather(gbuf, [rowj, gcol])
            g1 = plsc.load_gather(gbuf, [rowj, gcol + 1])
            g2 = plsc.load_gather(gbuf, [rowj, gcol + 2])
            v0 = g0 + ln0
            v1 = g1 + ln1
            v2 = g2 + ln2
            a = jnp.where(v2 > jnp.maximum(v0, v1), 2,
                          jnp.where(v1 > v0, 1, 0)).astype(jnp.int32)
            c = jnp.where(g2 + ls2 > g1 + ls1, 2, 1).astype(jnp.int32)
            samp = jnp.where((p1 == 2) & (p2 == 1), c, a)
            plsc.store_scatter(sbuf, [srow, rowj], samp)
            return (samp, p1)

        sbuf0[0, 0:_LANES] = jnp.ones((_LANES,), jnp.int32)
        ones = sbuf0[0, 0:_LANES]
        carry = (ones, ones)

        g_in = [None, None]
        s_out = [None, None]
        g_in[0] = pltpu.async_copy(
            g_hbm.at[pl.ds(0, _CHUNK)], gbufs[0], gsems[0])
        for ch in range(_NCHUNK):
            cur = ch % 2
            nxt = 1 - cur
            if ch + 1 < _NCHUNK:
                g_in[nxt] = pltpu.async_copy(
                    g_hbm.at[pl.ds((ch + 1) * _CHUNK, _CHUNK)],
                    gbufs[nxt], gsems[nxt])
            g_in[cur].wait()
            if s_out[cur] is not None:
                s_out[cur].wait()
            carry = lax.fori_loop(
                0, _CHUNK,
                functools.partial(step, gbuf=gbufs[cur], sbuf=sbufs[cur]),
                carry)
            s_out[cur] = pltpu.async_copy(
                sbufs[cur], s_hbm.at[cid, :, pl.ds(ch * _CHUNK, _CHUNK)],
                ssems[cur])
        s_out[0].wait()
        s_out[1].wait()


def _chain_samples(g):
    n1, B, _ = g.shape
    ntile = B // _LANES
    npad = _NCHUNK * _CHUNK
    g_flat = jnp.pad(g.reshape(n1, B * 3), ((0, npad - n1), (0, 0)))

    chain = pl.kernel(
        _chain_body,
        out_type=jax.ShapeDtypeStruct((ntile, _LANES, npad), jnp.int32),
        mesh=plsc.VectorSubcoreMesh(core_axis_name="c", subcore_axis_name="s"),
        scratch_types=[
            pltpu.VMEM((_CHUNK, 3 * B), jnp.float32),
            pltpu.VMEM((_CHUNK, 3 * B), jnp.float32),
            pltpu.VMEM((_LANES, _CHUNK), jnp.int32),
            pltpu.VMEM((_LANES, _CHUNK), jnp.int32),
            pltpu.SemaphoreType.DMA,
            pltpu.SemaphoreType.DMA,
            pltpu.SemaphoreType.DMA,
            pltpu.SemaphoreType.DMA,
        ],
        compiler_params=pltpu.CompilerParams(use_tc_tiling_on_sc=False,
                                             needs_layout_passes=False),
    )
    s = chain(g_flat)
    return s.reshape(B, npad)


def _select_kernel(s_ref, x_ref, o_ref):
    n1 = o_ref.shape[2] - 1
    s = s_ref[0, :, pl.ds(0, n1)]
    x1 = x_ref[0, :, pl.ds(1, n1)]
    x2 = x_ref[0, :, pl.ds(2, n1)]
    x3 = x_ref[0, :, pl.ds(3, n1)]
    o_ref[0, :, pl.ds(1, n1)] = jnp.where(s == 0, x1,
                                          jnp.where(s == 1, x2, x3))
    o_ref[0, :, 0:1] = x_ref[0, :, 1:2]


def kernel(x):
    B, I, T = x.shape
    n_win = T - 2
    n1 = n_win - 1

    keys = jax.random.split(jax.random.key(42), n1)
    g = jax.vmap(lambda k: jax.random.gumbel(k, (B, 3), jnp.float32))(keys)

    s = _chain_samples(g)
    s3 = s.reshape(B, 1, s.shape[1])

    ki = I // _ICHUNK
    out = pl.pallas_call(
        _select_kernel,
        grid=(B, ki),
        in_specs=[
            pl.BlockSpec((1, 1, s.shape[1]), lambda b, i: (b, 0, 0)),
            pl.BlockSpec((1, _ICHUNK, T), lambda b, i: (b, i, 0)),
        ],
        out_specs=pl.BlockSpec((1, _ICHUNK, n_win), lambda b, i: (b, i, 0)),
        out_shape=jax.ShapeDtypeStruct((B, I, n_win), x.dtype),
    )(s3, x)
    return out


# speedup vs baseline: 1.3361x; 1.3361x over previous
import functools
import numpy as np
import jax
import jax.numpy as jnp
from jax import lax
from jax.experimental import pallas as pl
from jax.experimental.pallas import tpu as pltpu
from jax.experimental.pallas import tpu_sc as plsc

_P = 0.1
_S = 1.0 - 2.0 * _P
_tmp = np.tile(np.array([_P, _S, _P], dtype=np.float32), (3, 3, 1))
_tmp[2, 1] = np.array([0.0, _S / (_P + _S), _P / (_P + _S)], dtype=np.float32)
_LOGITS = np.where(_tmp > 0, np.log(np.maximum(_tmp, 1e-30)), -1e30).astype(np.float32)
_LN = _LOGITS[0, 0]
_LS = _LOGITS[2, 1]

_LANES = 16
_CHUNK = 256
_NCHUNK = 16


def _chain_body(g_hbm, d_hbm, gbuf0, gbuf1, dbuf0, dbuf1,
                gsem0, gsem1, dsem0, dsem1):
    cid = lax.axis_index("c")
    sid = lax.axis_index("s")

    @pl.when(sid == 0)
    def _():
        gbufs = (gbuf0, gbuf1)
        dbufs = (dbuf0, dbuf1)
        gsems = (gsem0, gsem1)
        dsems = (dsem0, dsem1)

        ln0 = jnp.float32(_LN[0])
        ln1 = jnp.float32(_LN[1])
        ln2 = jnp.float32(_LN[2])
        ls1 = jnp.float32(_LS[1])
        ls2 = jnp.float32(_LS[2])

        def step(j, carry, gbuf, dbuf):
            p1, p2 = carry
            g0 = gbuf[j, 0]
            g1 = gbuf[j, 1]
            g2 = gbuf[j, 2]
            v0 = g0 + ln0
            v1 = g1 + ln1
            v2 = g2 + ln2
            a = jnp.where(v2 > jnp.maximum(v0, v1), 2,
                          jnp.where(v1 > v0, 1, 0)).astype(jnp.int32)
            c = jnp.where(g2 + ls2 > g1 + ls1, 2, 1).astype(jnp.int32)
            samp = jnp.where((p1 == 2) & (p2 == 1), c, a)
            dbuf[j] = samp
            return (samp, p1)

        dbuf0[0] = jnp.ones((_LANES,), jnp.int32)
        ones = dbuf0[0]
        carry = (ones, ones)

        g_in = [None, None]
        d_out = [None, None]
        g_in[0] = pltpu.async_copy(
            g_hbm.at[cid, pl.ds(0, _CHUNK)], gbufs[0], gsems[0])
        for ch in range(_NCHUNK):
            cur = ch % 2
            nxt = 1 - cur
            if ch + 1 < _NCHUNK:
                g_in[nxt] = pltpu.async_copy(
                    g_hbm.at[cid, pl.ds((ch + 1) * _CHUNK, _CHUNK)],
                    gbufs[nxt], gsems[nxt])
            g_in[cur].wait()
            if d_out[cur] is not None:
                d_out[cur].wait()
            carry = lax.fori_loop(
                1 if ch == 0 else 0, _CHUNK,
                functools.partial(step, gbuf=gbufs[cur], dbuf=dbufs[cur]),
                carry)
            d_out[cur] = pltpu.async_copy(
                dbufs[cur], d_hbm.at[cid, pl.ds(ch * _CHUNK, _CHUNK)],
                dsems[cur])
        d_out[0].wait()
        d_out[1].wait()


def _chain_offsets(g):
    n1, B, _ = g.shape
    ntile = B // _LANES
    npad = _NCHUNK * _CHUNK
    g_pad = jnp.pad(g, ((1, npad - n1 - 1), (0, 0), (0, 0)))
    g_sc = g_pad.reshape(npad, ntile, _LANES, 3).transpose(1, 0, 3, 2)

    chain = pl.kernel(
        _chain_body,
        out_type=jax.ShapeDtypeStruct((ntile, npad, _LANES), jnp.int32),
        mesh=plsc.VectorSubcoreMesh(core_axis_name="c", subcore_axis_name="s"),
        scratch_types=[
            pltpu.VMEM((_CHUNK, 3, _LANES), jnp.float32),
            pltpu.VMEM((_CHUNK, 3, _LANES), jnp.float32),
            pltpu.VMEM((_CHUNK, _LANES), jnp.int32),
            pltpu.VMEM((_CHUNK, _LANES), jnp.int32),
            pltpu.SemaphoreType.DMA,
            pltpu.SemaphoreType.DMA,
            pltpu.SemaphoreType.DMA,
            pltpu.SemaphoreType.DMA,
        ],
        compiler_params=pltpu.CompilerParams(use_tc_tiling_on_sc=False),
    )
    d = chain(g_sc)
    return d.transpose(0, 2, 1).reshape(B, npad)


def _select_kernel(d_ref, x_ref, o_ref):
    n = o_ref.shape[2]
    d = d_ref[0, :, pl.ds(0, n)]
    x0 = x_ref[0, :, pl.ds(0, n)]
    x1 = x_ref[0, :, pl.ds(1, n)]
    x2 = x_ref[0, :, pl.ds(2, n)]
    o_ref[0] = jnp.where(d == 0, x0, jnp.where(d == 1, x1, x2))


def kernel(x):
    B, I, T = x.shape
    n_win = T - 2
    n1 = n_win - 1

    keys = jax.random.split(jax.random.key(42), n1)
    g = jax.vmap(lambda k: jax.random.gumbel(k, (B, 3), jnp.float32))(keys)

    d = _chain_offsets(g)
    d3 = d.reshape(B, 1, d.shape[1])

    out = pl.pallas_call(
        _select_kernel,
        grid=(B,),
        in_specs=[
            pl.BlockSpec((1, 1, d.shape[1]), lambda b: (b, 0, 0)),
            pl.BlockSpec((1, I, T), lambda b: (b, 0, 0)),
        ],
        out_specs=pl.BlockSpec((1, I, n_win), lambda b: (b, 0, 0)),
        out_shape=jax.ShapeDtypeStruct((B, I, n_win), x.dtype),
    )(d3, x)
    return out


# speedup vs baseline: 1.3471x; 1.0082x over previous
import functools
import numpy as np
import jax
import jax.numpy as jnp
from jax import lax
from jax.experimental import pallas as pl
from jax.experimental.pallas import tpu as pltpu
from jax.experimental.pallas import tpu_sc as plsc

_P = 0.1
_S = 1.0 - 2.0 * _P
_tmp = np.tile(np.array([_P, _S, _P], dtype=np.float32), (3, 3, 1))
_tmp[2, 1] = np.array([0.0, _S / (_P + _S), _P / (_P + _S)], dtype=np.float32)
_LOGITS = np.where(_tmp > 0, np.log(np.maximum(_tmp, 1e-30)), -1e30).astype(np.float32)
_LN = _LOGITS[0, 0]
_LS = _LOGITS[2, 1]

_LANES = 16
_CHUNK = 256
_NCHUNK = 16


def _chain_body(g_hbm, d_hbm, gbuf0, gbuf1, dbuf0, dbuf1,
                gsem0, gsem1, dsem0, dsem1):
    cid = lax.axis_index("c")
    sid = lax.axis_index("s")

    @pl.when(sid == 0)
    def _():
        gbufs = (gbuf0, gbuf1)
        dbufs = (dbuf0, dbuf1)
        gsems = (gsem0, gsem1)
        dsems = (dsem0, dsem1)

        ln0 = jnp.float32(_LN[0])
        ln1 = jnp.float32(_LN[1])
        ln2 = jnp.float32(_LN[2])
        ls1 = jnp.float32(_LS[1])
        ls2 = jnp.float32(_LS[2])

        def step(j, carry, gbuf, dbuf):
            p1, p2 = carry
            g0 = gbuf[j, 0]
            g1 = gbuf[j, 1]
            g2 = gbuf[j, 2]
            v0 = g0 + ln0
            v1 = g1 + ln1
            v2 = g2 + ln2
            a = jnp.where(v2 > jnp.maximum(v0, v1), 2,
                          jnp.where(v1 > v0, 1, 0)).astype(jnp.int32)
            c = jnp.where(g2 + ls2 > g1 + ls1, 2, 1).astype(jnp.int32)
            samp = jnp.where((p1 == 2) & (p2 == 1), c, a)
            dbuf[j] = samp
            return (samp, p1)

        dbuf0[0] = jnp.ones((_LANES,), jnp.int32)
        ones = dbuf0[0]
        carry = (ones, ones)

        g_in = [None, None]
        d_out = [None, None]
        g_in[0] = pltpu.async_copy(
            g_hbm.at[cid, pl.ds(0, _CHUNK)], gbufs[0], gsems[0])
        for ch in range(_NCHUNK):
            cur = ch % 2
            nxt = 1 - cur
            if ch + 1 < _NCHUNK:
                g_in[nxt] = pltpu.async_copy(
                    g_hbm.at[cid, pl.ds((ch + 1) * _CHUNK, _CHUNK)],
                    gbufs[nxt], gsems[nxt])
            g_in[cur].wait()
            if d_out[cur] is not None:
                d_out[cur].wait()

            def step2(i, carry, gbuf, dbuf, base):
                j = base + 2 * i
                carry = step(j, carry, gbuf, dbuf)
                return step(j + 1, carry, gbuf, dbuf)

            if ch == 0:
                carry = step(1, carry, gbufs[cur], dbufs[cur])
                base, pairs = 2, (_CHUNK - 2) // 2
            else:
                base, pairs = 0, _CHUNK // 2
            carry = lax.fori_loop(
                0, pairs,
                functools.partial(step2, gbuf=gbufs[cur], dbuf=dbufs[cur],
                                  base=base),
                carry)
            d_out[cur] = pltpu.async_copy(
                dbufs[cur], d_hbm.at[cid, pl.ds(ch * _CHUNK, _CHUNK)],
                dsems[cur])
        d_out[0].wait()
        d_out[1].wait()


def _chain_offsets(g):
    n1, B, _ = g.shape
    ntile = B // _LANES
    npad = _NCHUNK * _CHUNK
    g_pad = jnp.pad(g, ((1, npad - n1 - 1), (0, 0), (0, 0)))
    g_sc = g_pad.reshape(npad, ntile, _LANES, 3).transpose(1, 0, 3, 2)

    chain = pl.kernel(
        _chain_body,
        out_type=jax.ShapeDtypeStruct((ntile, npad, _LANES), jnp.int32),
        mesh=plsc.VectorSubcoreMesh(core_axis_name="c", subcore_axis_name="s"),
        scratch_types=[
            pltpu.VMEM((_CHUNK, 3, _LANES), jnp.float32),
            pltpu.VMEM((_CHUNK, 3, _LANES), jnp.float32),
            pltpu.VMEM((_CHUNK, _LANES), jnp.int32),
            pltpu.VMEM((_CHUNK, _LANES), jnp.int32),
            pltpu.SemaphoreType.DMA,
            pltpu.SemaphoreType.DMA,
            pltpu.SemaphoreType.DMA,
            pltpu.SemaphoreType.DMA,
        ],
        compiler_params=pltpu.CompilerParams(use_tc_tiling_on_sc=False),
    )
    d = chain(g_sc)
    return d.transpose(0, 2, 1).reshape(B, npad)


def _select_kernel(d_ref, x_ref, o_ref):
    n = o_ref.shape[2]
    d = d_ref[0, :, pl.ds(0, n)]
    x0 = x_ref[0, :, pl.ds(0, n)]
    x1 = x_ref[0, :, pl.ds(1, n)]
    x2 = x_ref[0, :, pl.ds(2, n)]
    o_ref[0] = jnp.where(d == 0, x0, jnp.where(d == 1, x1, x2))


def kernel(x):
    B, I, T = x.shape
    n_win = T - 2
    n1 = n_win - 1

    keys = jax.random.split(jax.random.key(42), n1)
    g = jax.vmap(lambda k: jax.random.gumbel(k, (B, 3), jnp.float32))(keys)

    d = _chain_offsets(g)
    d3 = d.reshape(B, 1, d.shape[1])

    out = pl.pallas_call(
        _select_kernel,
        grid=(B,),
        in_specs=[
            pl.BlockSpec((1, 1, d.shape[1]), lambda b: (b, 0, 0)),
            pl.BlockSpec((1, I, T), lambda b: (b, 0, 0)),
        ],
        out_specs=pl.BlockSpec((1, I, n_win), lambda b: (b, 0, 0)),
        out_shape=jax.ShapeDtypeStruct((B, I, n_win), x.dtype),
    )(d3, x)
    return out
